# baseline (speedup 1.0000x reference)
import functools

import jax
import jax.numpy as jnp
from jax import lax
from jax.experimental import pallas as pl
from jax.experimental.pallas import tpu as pltpu
from jax.experimental.pallas import tpu_sc as plsc

F32 = jnp.float32
D = 128
SRCW = 144
DSTW = 80
AGGW = 144
NC, NS = 2, 16
K = 32
NBG = 4
NBM = 2
IDXG = 7
IDXD = 5
TC_ROWS = 1000


def _mm_t(x, w):
    return lax.dot_general(x, w, (((1,), (1,)), ((), ())),
                           preferred_element_type=F32)


def _pack16(lo, hi):
    bl = lax.bitcast_convert_type(lo, jnp.int32)
    bh = lax.bitcast_convert_type(hi, jnp.int32)
    bl = lax.shift_right_logical(bl + jnp.int32(0x8000), 16)
    bh = (bh + jnp.int32(0x8000)) & jnp.int32(-65536)
    return bl | bh


def _pack_block(x):
    cols = []
    for p in range(x.shape[1] // 32):
        cols.append(_pack16(x[:, 32 * p:32 * p + 16], x[:, 32 * p + 16:32 * p + 32]))
    return jnp.concatenate(cols, axis=1)


def _build_tables(x, p, src_ref, dst_ref):
    h = _mm_t(x, p["W"][...]) + p["Wb"][...]
    hU = _mm_t(h, p["U"][...]) + p["Ub"][...]
    hV = _mm_t(h, p["V"][...]) + p["Vb"][...]
    a_s = jnp.dot(h, p["A2p"][...], preferred_element_type=F32)
    a_d = jnp.dot(h, p["A1p"][...], preferred_element_type=F32) + p["Abp"][...]
    zz = jnp.zeros_like(a_s)
    src_ref[...] = jnp.concatenate(
        [_pack_block(h), _pack_block(hV), _pack16(a_s, zz)], axis=1)
    dst_ref[...] = jnp.concatenate([_pack_block(hU), _pack16(a_d, zz)], axis=1)


def _tables_body(x_ref, W, Wb, U, Ub, V, Vb, A1p, A2p, Abp, src_ref, dst_ref):
    p = {"W": W, "Wb": Wb, "U": U, "Ub": Ub, "V": V, "Vb": Vb,
         "A1p": A1p, "A2p": A2p, "Abp": Abp}
    _build_tables(x_ref[...], p, src_ref, dst_ref)


def _combine(p_ref, g, b):
    psum = p_ref[0] + p_ref[1]
    s = psum[:, 128:129]
    aggr = psum[:, :D] / (s + 1e-16)
    mu = jnp.mean(aggr, axis=-1, keepdims=True)
    var = jnp.mean((aggr - mu) ** 2, axis=-1, keepdims=True)
    return (aggr - mu) * lax.rsqrt(var + 1e-5) * g[...] + b[...]


def _combine_tables_body(p_ref, x_ref, g0, b0,
                         W, Wb, U, Ub, V, Vb, A1p, A2p, Abp,
                         src_ref, dst_ref):
    y = _combine(p_ref, g0, b0)
    x1 = jnp.maximum(y + x_ref[...], 0.0)
    p = {"W": W, "Wb": Wb, "U": U, "Ub": Ub, "V": V, "Vb": Vb,
         "A1p": A1p, "A2p": A2p, "Abp": Abp}
    _build_tables(x1, p, src_ref, dst_ref)


def _combine_final_body(p_ref, g1, b1, out_ref):
    out_ref[...] = _combine(p_ref, g1, b1)


def _row_spec(w):
    return pl.BlockSpec((TC_ROWS, w), lambda i: (i, 0))


def _full_spec(shape):
    nd = len(shape)
    return pl.BlockSpec(shape, lambda i, _n=nd: (0,) * _n)


def _prep_params(p):
    A = p["A"]
    A1 = A[0, :D]
    A2 = A[0, D:]
    A1p = jnp.zeros((D, 16), F32).at[:, 0].set(A1)
    A2p = jnp.zeros((D, 16), F32).at[:, 0].set(A2)
    Abp = jnp.zeros((1, 16), F32).at[0, 0].set(p["Ab"][0])
    return {"W": p["W"], "Wb": p["Wb"].reshape(1, D),
            "U": p["U"], "Ub": p["Ub"].reshape(1, D),
            "V": p["V"], "Vb": p["Vb"].reshape(1, D),
            "A1p": A1p, "A2p": A2p, "Abp": Abp,
            "g": p["ln_g"].reshape(1, D), "b": p["ln_b"].reshape(1, D)}


def _weight_args(q):
    ws = [q["W"], q["Wb"], q["U"], q["Ub"], q["V"], q["Vb"],
          q["A1p"], q["A2p"], q["Abp"]]
    return ws, [_full_spec(w.shape) for w in ws]


def _sc_edge_body(n_nodes, n_edges,
                  stab, dtab, sidx_h, didx_h, out,
                  sidx, didx, srows, drows, msg, semi, semg, sems, aggr):
    npad = ((n_nodes + NS * K - 1) // (NS * K)) * (NS * K)
    rpt = npad // NS
    c = lax.axis_index("c")
    s = lax.axis_index("s")
    wid = c * NS + s
    tot = n_edges // K
    ebase = (tot * wid) // (NC * NS)
    nchunk = (tot * (wid + 1)) // (NC * NS) - ebase

    zeros16 = jnp.zeros((16,), F32)
    iota = lax.iota(jnp.int32, 16)

    for m in range(NBM):
        def _zm(j, _, _m=m):
            for gcol in range(AGGW // 16):
                msg[_m, j, pl.ds(gcol * 16, 16)] = zeros16
            return 0
        lax.fori_loop(0, K, _zm, 0)
    for i in range(rpt // K):
        pltpu.sync_copy(msg.at[0], aggr.at[pl.ds(s * rpt + i * K, K)])
    plsc.subcore_barrier()

    G = IDXG
    ngroups = (nchunk + G - 1) // G

    def _idx_load(q, sync=False):
        gs = q % 2
        if sync:
            pltpu.sync_copy(sidx_h.at[pl.ds(ebase + q * G, G)], sidx.at[gs])
            pltpu.sync_copy(didx_h.at[pl.ds(ebase + q * G, G)], didx.at[gs])
        else:
            pltpu.async_copy(sidx_h.at[pl.ds(ebase + q * G, G)],
                             sidx.at[gs], semi.at[gs])
            pltpu.async_copy(didx_h.at[pl.ds(ebase + q * G, G)],
                             didx.at[gs], semi.at[gs])

    def _idx_wait(q):
        gs = q % 2
        pltpu.make_async_copy(sidx_h.at[pl.ds(ebase + q * G, G)],
                              sidx.at[gs], semi.at[gs]).wait()
        pltpu.make_async_copy(didx_h.at[pl.ds(ebase + q * G, G)],
                              didx.at[gs], semi.at[gs]).wait()

    def _srow(idx, ci):
        return idx.at[(ci // G) % 2, ci % G]

    col_as = jnp.full((16,), 128, jnp.int32)
    col_ad = jnp.full((16,), 64, jnp.int32)
    col_ex = jnp.full((16,), D, jnp.int32)

    def _flo(w):
        return plsc.bitcast(w << 16, F32)

    def _fhi(w):
        return plsc.bitcast(w, F32)

    def _issue(ci, b):
        pltpu.async_copy(stab.at[_srow(sidx, ci)], srows.at[b], semg.at[b])
        pltpu.async_copy(dtab.at[_srow(didx, ci)], drows.at[b], semg.at[b])

    def _wait_gather(ci, b):
        pltpu.make_async_copy(stab.at[_srow(sidx, ci)], srows.at[b],
                              semg.at[b]).wait()
        pltpu.make_async_copy(dtab.at[_srow(didx, ci)], drows.at[b],
                              semg.at[b]).wait()

    def _compute(ci, b, bm):
        sr = srows.at[b]
        dr = drows.at[b]
        mg = msg.at[bm]
        for gg in range(K // 16):
            rows = iota + 16 * gg
            a_s = _flo(plsc.load_gather(sr, [rows, col_as]))
            a_d = _flo(plsc.load_gather(dr, [rows, col_ad]))
            logit = a_s + a_d
            ex = jnp.exp(jnp.maximum(logit, logit * 0.2))
            plsc.store_scatter(mg, [rows, col_ex], ex)

        @plsc.parallel_loop(0, K, 1, unroll=4)
        def _edge(j):
            jv = jnp.full((16,), j, jnp.int32)
            exv = plsc.load_gather(mg, [jv, col_ex])
            wu = [dr[j, pl.ds(16 * g, 16)] for g in range(4)]
            wv = [sr[j, pl.ds(64 + 16 * g, 16)] for g in range(4)]
            ts = []
            for g in range(4):
                ts.append(_flo(wu[g]) + _flo(wv[g]))
                ts.append(_fhi(wu[g]) + _fhi(wv[g]))
            es = [jnp.exp(-t) for t in ts]
            rs = [1.0 / (1.0 + e) for e in es]
            for g in range(4):
                wh = sr[j, pl.ds(16 * g, 16)]
                mg[j, pl.ds(32 * g, 16)] = (exv * _flo(wh)) * rs[2 * g]
                mg[j, pl.ds(32 * g + 16, 16)] = (exv * _fhi(wh)) * rs[2 * g + 1]
        pltpu.async_copy(msg.at[bm], aggr.at[_srow(didx, ci)], sems.at[bm],
                         add=True)

    def _wait_scatter(ci, bm):
        pltpu.make_async_copy(msg.at[bm], aggr.at[_srow(didx, ci)],
                              sems.at[bm]).wait()

    _idx_load(0, sync=True)
    for p in range(NBG - 1):
        _issue(p, p)

    def _step(ci, _):
        b = ci % NBG
        bm = ci % NBM
        nxt = ci + (NBG - 1)
        _wait_gather(ci, b)

        @pl.when((nxt < nchunk) & (lax.rem(nxt, G) == 0))
        def _():
            _idx_wait(nxt // G)

        @pl.when((nxt < nchunk) & (lax.rem(nxt, G) == IDXD))
        def _():
            q = nxt // G

            @pl.when(q + 1 < ngroups)
            def _():
                _idx_load(q + 1)

        @pl.when(nxt < nchunk)
        def _():
            _issue(nxt, nxt % NBG)

        @pl.when(ci >= NBM)
        def _():
            _wait_scatter(ci - NBM, bm)
        _compute(ci, b, bm)
        return 0
    lax.fori_loop(0, nchunk, _step, 0)
    for t in range(NBM):
        ci = nchunk - NBM + t
        _wait_scatter(ci, ci % NBM)

    plsc.subcore_barrier()
    full = n_nodes // rpt
    rem = n_nodes - full * rpt

    @pl.when(s < full)
    def _():
        pltpu.sync_copy(aggr.at[pl.ds(s * rpt, rpt)],
                        out.at[c, pl.ds(s * rpt, rpt)])
    if rem:
        @pl.when(s == full)
        def _():
            pltpu.sync_copy(aggr.at[pl.ds(full * rpt, rem)],
                            out.at[c, pl.ds(full * rpt, rem)])


def _pad_idx(idx):
    n_edges = idx.shape[0]
    pad_rows = n_edges // K + 2 * IDXG
    zpad = jnp.zeros((pad_rows * K - n_edges,), jnp.int32)
    return jnp.concatenate([idx, zpad]).reshape(-1, K)


def _sc_edge(src_tab, dst_tab, src_idx2, dst_idx2, n_edges):
    n_nodes = src_tab.shape[0]
    npad = ((n_nodes + NS * K - 1) // (NS * K)) * (NS * K)
    mesh = plsc.VectorSubcoreMesh(core_axis_name="c", subcore_axis_name="s")
    run = pl.kernel(
        functools.partial(_sc_edge_body, n_nodes, n_edges),
        out_type=jax.ShapeDtypeStruct((NC, n_nodes, AGGW), F32),
        mesh=mesh,
        compiler_params=pltpu.CompilerParams(use_tc_tiling_on_sc=False,
                                             needs_layout_passes=False),
        scratch_types=[
            pltpu.VMEM((2, IDXG, K), jnp.int32),
            pltpu.VMEM((2, IDXG, K), jnp.int32),
            pltpu.VMEM((NBG, K, SRCW), jnp.int32),
            pltpu.VMEM((NBG, K, DSTW), jnp.int32),
            pltpu.VMEM((NBM, K, AGGW), F32),
            pltpu.SemaphoreType.DMA((2,)),
            pltpu.SemaphoreType.DMA((NBG,)),
            pltpu.SemaphoreType.DMA((NBM,)),
            pltpu.VMEM_SHARED((npad, AGGW), F32),
        ],
    )
    return run(src_tab, dst_tab, src_idx2, dst_idx2)


def kernel(x, edge_index, params):
    n = x.shape[0]
    grid = (n // TC_ROWS,)
    n_edges = edge_index.shape[1]
    src_idx2 = _pad_idx(edge_index[0])
    dst_idx2 = _pad_idx(edge_index[1])
    q0 = _prep_params(params["l0"])
    q1 = _prep_params(params["l1"])

    tab_shapes = [jax.ShapeDtypeStruct((n, SRCW), jnp.int32),
                  jax.ShapeDtypeStruct((n, DSTW), jnp.int32)]
    tab_specs = [_row_spec(SRCW), _row_spec(DSTW)]

    w0, w0_specs = _weight_args(q0)
    stab0, dtab0 = pl.pallas_call(
        _tables_body,
        grid=grid,
        in_specs=[_row_spec(D)] + w0_specs,
        out_specs=tab_specs,
        out_shape=tab_shapes,
    )(x, *w0)

    part0 = _sc_edge(stab0, dtab0, src_idx2, dst_idx2, n_edges)

    w1, w1_specs = _weight_args(q1)
    stab1, dtab1 = pl.pallas_call(
        _combine_tables_body,
        grid=grid,
        in_specs=[pl.BlockSpec((NC, TC_ROWS, AGGW), lambda i: (0, i, 0)),
                  _row_spec(D), _full_spec((1, D)), _full_spec((1, D))]
                 + w1_specs,
        out_specs=tab_specs,
        out_shape=tab_shapes,
    )(part0, x, q0["g"], q0["b"], *w1)

    part1 = _sc_edge(stab1, dtab1, src_idx2, dst_idx2, n_edges)

    out = pl.pallas_call(
        _combine_final_body,
        grid=grid,
        in_specs=[pl.BlockSpec((NC, TC_ROWS, AGGW), lambda i: (0, i, 0)),
                  _full_spec((1, D)), _full_spec((1, D))],
        out_specs=_row_spec(D),
        out_shape=jax.ShapeDtypeStruct((n, D), F32),
    )(part1, q1["g"], q1["b"])
    return out
